# baseline (speedup 1.0000x reference)
import functools

import jax
import jax.numpy as jnp
from jax import lax
from jax.experimental import pallas as pl
from jax.experimental.pallas import tpu as pltpu
from jax.experimental.pallas import tpu_sc as plsc


def _k_chunks(K):
    if K == 8192:
        return (0, 2736, 5472, 8192)
    return (0, K)


def _stage_tc(r, x2, cb_chunks, c2_chunks, offs):
    N, D = r.shape
    NT = 512 if N % 512 == 0 else N
    n_chunks = len(cb_chunks)

    def body(*refs):
        r_ref, x2_ref = refs[0], refs[1]
        cb_refs = refs[2:2 + n_chunks]
        c2_refs = refs[2 + n_chunks:2 + 2 * n_chunks]
        idx_ref = refs[2 + 2 * n_chunks]

        rbf = r_ref[...].astype(jnp.bfloat16)
        x2v = x2_ref[...]
        acc_v = None
        acc_i = None
        for c in range(n_chunks):
            mm = lax.dot_general(rbf, cb_refs[c][...],
                                 (((1,), (0,)), ((), ())),
                                 preferred_element_type=jnp.float32)
            d2 = jnp.maximum((x2v + c2_refs[c][...]) - 2.0 * mm, 0.0)
            s = jnp.sqrt(d2)
            cmin = jnp.min(s, axis=1, keepdims=True)
            kio = lax.broadcasted_iota(jnp.int32, s.shape, 1) + offs[c]
            cidx = jnp.min(jnp.where(s == cmin, kio, jnp.int32(2 ** 30)),
                           axis=1, keepdims=True)
            if c == 0:
                acc_v = cmin.astype(jnp.bfloat16)
                acc_i = cidx
            else:
                upd = cmin < acc_v.astype(jnp.float32)
                acc_i = jnp.where(upd, cidx, acc_i)
                acc_v = jnp.where(upd, cmin.astype(jnp.bfloat16), acc_v)
        idx_ref[...] = acc_i

    in_specs = [
        pl.BlockSpec((NT, D), lambda i: (i, 0)),
        pl.BlockSpec((NT, 1), lambda i: (i, 0)),
    ]
    for cb in cb_chunks:
        in_specs.append(pl.BlockSpec(cb.shape, lambda i: (0, 0)))
    for c2 in c2_chunks:
        in_specs.append(pl.BlockSpec(c2.shape, lambda i: (0, 0)))

    return pl.pallas_call(
        body,
        grid=(N // NT,),
        in_specs=in_specs,
        out_specs=pl.BlockSpec((NT, 1), lambda i: (i, 0)),
        out_shape=jax.ShapeDtypeStruct((N, 1), jnp.int32),
        compiler_params=pltpu.CompilerParams(
            dimension_semantics=("arbitrary",)),
    )(r, x2, *cb_chunks, *c2_chunks)


def _sc_gather(table, idx):
    K, D = table.shape
    N = idx.shape[0]
    info = plsc.get_sparse_core_info()
    NC, NS = info.num_cores, info.num_subcores
    NW = NC * NS
    b_per_w = N // NW
    CH = min(128, b_per_w)
    n_chunks = b_per_w // CH
    mesh = plsc.VectorSubcoreMesh(core_axis_name="c", subcore_axis_name="s")

    nbuf = min(3, n_chunks)
    idx2d = idx.reshape(N // CH, CH)

    @functools.partial(
        pl.kernel,
        out_type=jax.ShapeDtypeStruct((N, D), jnp.float32),
        mesh=mesh,
        scratch_types=[
            pltpu.VMEM((n_chunks, CH), jnp.int32),
            pltpu.VMEM((nbuf, CH, D), jnp.float32),
            pltpu.SemaphoreType.DMA,
            pltpu.SemaphoreType.DMA,
        ],
    )
    def k(table_hbm, idx_hbm, out_hbm, idx_v, rows_v, gsem, wsem):
        wid = lax.axis_index("s") * NC + lax.axis_index("c")
        base = wid * b_per_w
        pltpu.sync_copy(idx_hbm.at[pl.ds(wid * n_chunks, n_chunks)], idx_v)
        gathers = [None] * n_chunks
        writes = [None] * n_chunks
        for j in range(min(nbuf, n_chunks)):
            gathers[j] = pltpu.async_copy(
                table_hbm.at[idx_v.at[j]], rows_v.at[j], gsem)
        for j in range(n_chunks):
            gathers[j].wait()
            writes[j] = pltpu.async_copy(
                rows_v.at[j % nbuf], out_hbm.at[pl.ds(base + j * CH, CH)],
                wsem)
            nxt = j + nbuf
            if nxt < n_chunks:
                writes[j].wait()
                gathers[nxt] = pltpu.async_copy(
                    table_hbm.at[idx_v.at[nxt]], rows_v.at[nxt % nbuf], gsem)
        for j in range(max(0, n_chunks - nbuf), n_chunks):
            writes[j].wait()

    return k(table, idx2d)


def kernel(x, codebooks):
    N, D = x.shape
    C, K, _ = codebooks.shape
    edges = _k_chunks(K)
    offs = edges[:-1]
    cbt_bf = codebooks.transpose(0, 2, 1).astype(jnp.bfloat16)
    c2 = jnp.sum(codebooks * codebooks, axis=2)[:, None, :]

    r = x
    idxs, qs = [], []
    for i in range(C):
        cb_chunks = tuple(cbt_bf[i, :, a:b] for a, b in zip(edges[:-1], edges[1:]))
        c2_chunks = tuple(c2[i, :, a:b] for a, b in zip(edges[:-1], edges[1:]))
        x2 = jnp.sum(r * r, axis=1, keepdims=True)
        idx_i = _stage_tc(r, x2, cb_chunks, c2_chunks, offs)
        q = _sc_gather(codebooks[i], idx_i.reshape(N))
        idxs.append(idx_i.reshape(N))
        qs.append(q)
        if i + 1 < C:
            r = r - q

    out = functools.reduce(jnp.add, qs)
    return out, jnp.stack(idxs, axis=1)


# speedup vs baseline: 1.2067x; 1.2067x over previous
import functools

import jax
import jax.numpy as jnp
from jax import lax
from jax.experimental import pallas as pl
from jax.experimental.pallas import tpu as pltpu
from jax.experimental.pallas import tpu_sc as plsc


def _k_chunks(K):
    if K == 8192:
        return (0, 2736, 5472, 8192)
    return (0, K)


def _stage_tc(r, x2, cb_chunks, c2_chunks, offs):
    N, D = r.shape
    NT = 512 if N % 512 == 0 else N
    n_chunks = len(cb_chunks)

    def body(*refs):
        r_ref, x2_ref = refs[0], refs[1]
        cb_refs = refs[2:2 + n_chunks]
        c2_refs = refs[2 + n_chunks:2 + 2 * n_chunks]
        idx_ref = refs[2 + 2 * n_chunks]

        rbf = r_ref[...].astype(jnp.bfloat16)
        x2v = x2_ref[...]
        acc_v = None
        acc_i = None
        for c in range(n_chunks):
            mm2 = lax.dot_general(rbf, cb_refs[c][...],
                                  (((1,), (0,)), ((), ())),
                                  preferred_element_type=jnp.float32)
            t = (x2v + c2_refs[c][...]) - mm2
            tmin = jnp.min(t, axis=1, keepdims=True)
            smin = jnp.sqrt(jnp.maximum(tmin, 0.0))
            h0 = smin * smin
            bits = lax.bitcast_convert_type(h0, jnp.int32)
            bits = jnp.where(jnp.sqrt(h0) == smin, bits, bits - 1)
            h = lax.bitcast_convert_type(bits, jnp.float32)
            for j in range(1, 5):
                hj = lax.bitcast_convert_type(bits + j, jnp.float32)
                h = jnp.where(jnp.sqrt(hj) == smin, hj, h)
            kio = lax.broadcasted_iota(jnp.int32, t.shape, 1) + offs[c]
            cidx = jnp.min(jnp.where(t <= h, kio, jnp.int32(2 ** 30)),
                           axis=1, keepdims=True)
            if c == 0:
                acc_v = smin.astype(jnp.bfloat16)
                acc_i = cidx
            else:
                upd = smin < acc_v.astype(jnp.float32)
                acc_i = jnp.where(upd, cidx, acc_i)
                acc_v = jnp.where(upd, smin.astype(jnp.bfloat16), acc_v)
        idx_ref[...] = acc_i

    in_specs = [
        pl.BlockSpec((NT, D), lambda i: (i, 0)),
        pl.BlockSpec((NT, 1), lambda i: (i, 0)),
    ]
    for cb in cb_chunks:
        in_specs.append(pl.BlockSpec(cb.shape, lambda i: (0, 0)))
    for c2 in c2_chunks:
        in_specs.append(pl.BlockSpec(c2.shape, lambda i: (0, 0)))

    return pl.pallas_call(
        body,
        grid=(N // NT,),
        in_specs=in_specs,
        out_specs=pl.BlockSpec((NT, 1), lambda i: (i, 0)),
        out_shape=jax.ShapeDtypeStruct((N, 1), jnp.int32),
        compiler_params=pltpu.CompilerParams(
            dimension_semantics=("parallel",)),
    )(r, x2, *cb_chunks, *c2_chunks)


def _sc_gather(table, idx):
    K, D = table.shape
    N = idx.shape[0]
    info = plsc.get_sparse_core_info()
    NC, NS = info.num_cores, info.num_subcores
    NW = NC * NS
    b_per_w = N // NW
    CH = min(128, b_per_w)
    n_chunks = b_per_w // CH
    mesh = plsc.VectorSubcoreMesh(core_axis_name="c", subcore_axis_name="s")

    nbuf = min(3, n_chunks)
    idx2d = idx.reshape(N // CH, CH)

    @functools.partial(
        pl.kernel,
        out_type=jax.ShapeDtypeStruct((N, D), jnp.float32),
        mesh=mesh,
        scratch_types=[
            pltpu.VMEM((n_chunks, CH), jnp.int32),
            pltpu.VMEM((nbuf, CH, D), jnp.float32),
            pltpu.SemaphoreType.DMA,
            pltpu.SemaphoreType.DMA,
        ],
    )
    def k(table_hbm, idx_hbm, out_hbm, idx_v, rows_v, gsem, wsem):
        wid = lax.axis_index("s") * NC + lax.axis_index("c")
        base = wid * b_per_w
        pltpu.sync_copy(idx_hbm.at[pl.ds(wid * n_chunks, n_chunks)], idx_v)
        gathers = [None] * n_chunks
        writes = [None] * n_chunks
        for j in range(min(nbuf, n_chunks)):
            gathers[j] = pltpu.async_copy(
                table_hbm.at[idx_v.at[j]], rows_v.at[j], gsem)
        for j in range(n_chunks):
            gathers[j].wait()
            writes[j] = pltpu.async_copy(
                rows_v.at[j % nbuf], out_hbm.at[pl.ds(base + j * CH, CH)],
                wsem)
            nxt = j + nbuf
            if nxt < n_chunks:
                writes[j].wait()
                gathers[nxt] = pltpu.async_copy(
                    table_hbm.at[idx_v.at[nxt]], rows_v.at[nxt % nbuf], gsem)
        for j in range(max(0, n_chunks - nbuf), n_chunks):
            writes[j].wait()

    return k(table, idx2d)


def kernel(x, codebooks):
    N, D = x.shape
    C, K, _ = codebooks.shape
    edges = _k_chunks(K)
    offs = edges[:-1]
    cbt_bf = codebooks.transpose(0, 2, 1).astype(jnp.bfloat16) * jnp.bfloat16(2)
    c2 = jnp.sum(codebooks * codebooks, axis=2)[:, None, :]

    r = x
    idxs, qs = [], []
    for i in range(C):
        cb_chunks = tuple(cbt_bf[i, :, a:b] for a, b in zip(edges[:-1], edges[1:]))
        c2_chunks = tuple(c2[i, :, a:b] for a, b in zip(edges[:-1], edges[1:]))
        x2 = jnp.sum(r * r, axis=1, keepdims=True)
        idx_i = _stage_tc(r, x2, cb_chunks, c2_chunks, offs)
        q = _sc_gather(codebooks[i], idx_i.reshape(N))
        idxs.append(idx_i.reshape(N))
        qs.append(q)
        if i + 1 < C:
            r = r - q

    out = functools.reduce(jnp.add, qs)
    return out, jnp.stack(idxs, axis=1)


# speedup vs baseline: 1.6191x; 1.3418x over previous
import functools

import jax
import jax.numpy as jnp
from jax import lax
from jax.experimental import pallas as pl
from jax.experimental.pallas import tpu as pltpu
from jax.experimental.pallas import tpu_sc as plsc


def _k_chunks(K):
    if K == 8192:
        return (0, 2736, 5472, 8192)
    return (0, K)


def _stage_tc(r, x2, cb_chunks, c2_chunks, offs):
    N, D = r.shape
    NT = 512 if N % 512 == 0 else N
    n_chunks = len(cb_chunks)

    def body(*refs):
        r_ref, x2_ref = refs[0], refs[1]
        cb_refs = refs[2:2 + n_chunks]
        c2_refs = refs[2 + n_chunks:2 + 2 * n_chunks]
        idx_ref = refs[2 + 2 * n_chunks]

        rbf = r_ref[...].astype(jnp.bfloat16)
        x2v = x2_ref[...]
        acc_v = None
        acc_i = None
        for c in range(n_chunks):
            mm2 = lax.dot_general(rbf, cb_refs[c][...],
                                  (((1,), (0,)), ((), ())),
                                  preferred_element_type=jnp.float32)
            t = (x2v + c2_refs[c][...]) - mm2
            tmin = jnp.min(t, axis=1, keepdims=True)
            smin = jnp.sqrt(jnp.maximum(tmin, 0.0))
            h0 = smin * smin
            bits = lax.bitcast_convert_type(h0, jnp.int32)
            bits = jnp.where(jnp.sqrt(h0) == smin, bits, bits - 1)
            h = lax.bitcast_convert_type(bits, jnp.float32)
            for j in range(1, 5):
                hj = lax.bitcast_convert_type(bits + j, jnp.float32)
                h = jnp.where(jnp.sqrt(hj) == smin, hj, h)
            kio = lax.broadcasted_iota(jnp.int32, t.shape, 1) + offs[c]
            cidx = jnp.min(jnp.where(t <= h, kio, jnp.int32(2 ** 30)),
                           axis=1, keepdims=True)
            if c == 0:
                acc_v = smin.astype(jnp.bfloat16)
                acc_i = cidx
            else:
                upd = smin < acc_v.astype(jnp.float32)
                acc_i = jnp.where(upd, cidx, acc_i)
                acc_v = jnp.where(upd, smin.astype(jnp.bfloat16), acc_v)
        idx_ref[...] = acc_i

    in_specs = [
        pl.BlockSpec((NT, D), lambda i: (i, 0)),
        pl.BlockSpec((NT, 1), lambda i: (i, 0)),
    ]
    for cb in cb_chunks:
        in_specs.append(pl.BlockSpec(cb.shape, lambda i: (0, 0)))
    for c2 in c2_chunks:
        in_specs.append(pl.BlockSpec(c2.shape, lambda i: (0, 0)))

    return pl.pallas_call(
        body,
        grid=(N // NT,),
        in_specs=in_specs,
        out_specs=pl.BlockSpec((NT, 1), lambda i: (i, 0)),
        out_shape=jax.ShapeDtypeStruct((N, 1), jnp.int32),
        compiler_params=pltpu.CompilerParams(
            dimension_semantics=("parallel",)),
    )(r, x2, *cb_chunks, *c2_chunks)


def _sc_gather(table, idx):
    K, D = table.shape
    N = idx.shape[0]
    info = plsc.get_sparse_core_info()
    NC, NS = info.num_cores, info.num_subcores
    NW = NC * NS
    b_per_w = N // NW
    CH = min(128, b_per_w)
    n_chunks = b_per_w // CH
    mesh = plsc.VectorSubcoreMesh(core_axis_name="c", subcore_axis_name="s")

    nbuf = min(3, n_chunks)
    idx2d = idx.reshape(N // CH, CH)

    @functools.partial(
        pl.kernel,
        out_type=jax.ShapeDtypeStruct((N, D), jnp.float32),
        mesh=mesh,
        scratch_types=[
            pltpu.VMEM((n_chunks, CH), jnp.int32),
            pltpu.VMEM((nbuf, CH, D), jnp.float32),
            pltpu.SemaphoreType.DMA,
            pltpu.SemaphoreType.DMA,
        ],
    )
    def k(table_hbm, idx_hbm, out_hbm, idx_v, rows_v, gsem, wsem):
        wid = lax.axis_index("s") * NC + lax.axis_index("c")
        base = wid * b_per_w
        pltpu.sync_copy(idx_hbm.at[pl.ds(wid * n_chunks, n_chunks)], idx_v)
        gathers = [None] * n_chunks
        writes = [None] * n_chunks
        for j in range(min(nbuf, n_chunks)):
            gathers[j] = pltpu.async_copy(
                table_hbm.at[idx_v.at[j]], rows_v.at[j], gsem)
        for j in range(n_chunks):
            gathers[j].wait()
            writes[j] = pltpu.async_copy(
                rows_v.at[j % nbuf], out_hbm.at[pl.ds(base + j * CH, CH)],
                wsem)
            nxt = j + nbuf
            if nxt < n_chunks:
                writes[j].wait()
                gathers[nxt] = pltpu.async_copy(
                    table_hbm.at[idx_v.at[nxt]], rows_v.at[nxt % nbuf], gsem)
        for j in range(max(0, n_chunks - nbuf), n_chunks):
            writes[j].wait()

    return k(table, idx2d)


def kernel(x, codebooks):
    N, D = x.shape
    C, K, _ = codebooks.shape
    edges = _k_chunks(K)
    offs = edges[:-1]
    cbt_bf = codebooks.transpose(0, 2, 1).astype(jnp.bfloat16) * jnp.bfloat16(2)
    c2 = jnp.sum(codebooks * codebooks, axis=2)[:, None, :]

    H = N // 2
    halves = [x[:H], x[H:]]
    idxs = [[], []]
    qs = [[], []]
    for i in range(C):
        cb_chunks = tuple(cbt_bf[i, :, a:b] for a, b in zip(edges[:-1], edges[1:]))
        c2_chunks = tuple(c2[i, :, a:b] for a, b in zip(edges[:-1], edges[1:]))
        for h in range(2):
            r = halves[h]
            x2 = jnp.sum(r * r, axis=1, keepdims=True)
            idx_i = _stage_tc(r, x2, cb_chunks, c2_chunks, offs)
            q = _sc_gather(codebooks[i], idx_i.reshape(H))
            idxs[h].append(idx_i.reshape(H))
            qs[h].append(q)
            if i + 1 < C:
                halves[h] = r - q

    out = jnp.concatenate(
        [functools.reduce(jnp.add, qs[0]), functools.reduce(jnp.add, qs[1])])
    return out, jnp.concatenate(
        [jnp.stack(idxs[0], axis=1), jnp.stack(idxs[1], axis=1)])
